# baseline (speedup 1.0000x reference)
import functools

import jax
import jax.numpy as jnp
from jax import lax
from jax.experimental import pallas as pl
from jax.experimental.pallas import tpu as pltpu
from jax.experimental.pallas import tpu_sc as plsc

_NC = 2
_NS = 16
_NW = _NC * _NS
_NBUF = 8


@jax.jit
def _embedding_gather(idx, weight):
    B, H = idx.shape
    D = weight.shape[1]
    n_chunks = B // _NW
    mesh = plsc.VectorSubcoreMesh(core_axis_name="c", subcore_axis_name="s")

    @functools.partial(
        pl.kernel,
        mesh=mesh,
        out_type=jax.ShapeDtypeStruct((B, H, D), jnp.float32),
        scratch_types=[
            pltpu.VMEM((n_chunks, H), jnp.int32),
            pltpu.VMEM((_NBUF, H, D), jnp.float32),
            pltpu.SemaphoreType.DMA((_NBUF,)),
            pltpu.SemaphoreType.DMA((_NBUF,)),
        ],
        compiler_params=pltpu.CompilerParams(use_tc_tiling_on_sc=True),
    )
    def k(idx_hbm, table_hbm, out_hbm, idx_v, rows_v, gsem, ssem):
        wid = lax.axis_index("s") * _NC + lax.axis_index("c")
        base = wid * n_chunks

        pltpu.sync_copy(idx_hbm.at[pl.ds(base, n_chunks)], idx_v)

        def gather(j, b):
            pltpu.async_copy(table_hbm.at[idx_v.at[j]], rows_v.at[b], gsem.at[b])

        def gather_wait(j, b):
            pltpu.make_async_copy(
                table_hbm.at[idx_v.at[j]], rows_v.at[b], gsem.at[b]
            ).wait()

        def scatter(j, b):
            pltpu.async_copy(rows_v.at[b], out_hbm.at[base + j], ssem.at[b])

        def scatter_wait(b):
            pltpu.make_async_copy(
                rows_v.at[b], out_hbm.at[base], ssem.at[b]
            ).wait()

        for b in range(_NBUF - 1):
            gather(b, b)

        @pl.loop(0, n_chunks, step=_NBUF)
        def _(j0):
            for b in range(_NBUF):
                j = j0 + b
                pb = (b - 1) % _NBUF
                gather_wait(j, b)
                scatter(j, b)
                @pl.when(j > 0)
                def _():
                    scatter_wait(pb)

                @pl.when(j + _NBUF - 1 < n_chunks)
                def _():
                    gather(j + _NBUF - 1, pb)

        scatter_wait((n_chunks - 1) % _NBUF)

    return k(idx, weight)


def kernel(input_ids, weight):
    return _embedding_gather(input_ids.astype(jnp.int32), weight)


# speedup vs baseline: 1.7592x; 1.7592x over previous
import functools

import jax
import jax.numpy as jnp
from jax import lax
from jax.experimental import pallas as pl
from jax.experimental.pallas import tpu as pltpu
from jax.experimental.pallas import tpu_sc as plsc

_NC = 2
_NS = 16
_NW = _NC * _NS
_CHUNK = 128
_NBUF = 5


@jax.jit
def _embedding_gather(idx, weight):
    n_chunks = idx.shape[1]
    D = weight.shape[1]
    R = _NW * n_chunks * _CHUNK
    rows_per_w = n_chunks * _CHUNK
    mesh = plsc.VectorSubcoreMesh(core_axis_name="c", subcore_axis_name="s")

    @functools.partial(
        pl.kernel,
        mesh=mesh,
        out_type=jax.ShapeDtypeStruct((R, D), jnp.float32),
        scratch_types=[
            pltpu.VMEM((n_chunks, _CHUNK), jnp.int32),
            pltpu.VMEM((_NBUF, _CHUNK, D), jnp.float32),
            pltpu.SemaphoreType.DMA((_NBUF,)),
            pltpu.SemaphoreType.DMA((_NBUF,)),
        ],
    )
    def k(idx_hbm, table_hbm, out_hbm, idx_v, rows_v, gsem, ssem):
        wid = lax.axis_index("s") * _NC + lax.axis_index("c")
        base = wid * rows_per_w

        pltpu.sync_copy(idx_hbm.at[wid], idx_v)

        def gather(j, b):
            pltpu.async_copy(table_hbm.at[idx_v.at[j]], rows_v.at[b], gsem.at[b])

        def gather_wait(j, b):
            pltpu.make_async_copy(
                table_hbm.at[idx_v.at[j]], rows_v.at[b], gsem.at[b]
            ).wait()

        def scatter(j, b):
            pltpu.async_copy(
                rows_v.at[b], out_hbm.at[pl.ds(base + j * _CHUNK, _CHUNK)], ssem.at[b]
            )

        def scatter_wait(b):
            pltpu.make_async_copy(
                rows_v.at[b], out_hbm.at[pl.ds(base, _CHUNK)], ssem.at[b]
            ).wait()

        for b in range(_NBUF - 1):
            gather(b, b)

        @pl.loop(0, n_chunks, step=_NBUF)
        def _(j0):
            for b in range(_NBUF):
                j = j0 + b
                pb = (b - 1) % _NBUF
                gather_wait(j, b)
                scatter(j, b)
                @pl.when(j > 0)
                def _():
                    scatter_wait(pb)

                @pl.when(j + _NBUF - 1 < n_chunks)
                def _():
                    gather(j + _NBUF - 1, pb)

        scatter_wait((n_chunks - 1) % _NBUF)

    return k(idx, weight)


def kernel(input_ids, weight):
    B, H = input_ids.shape
    D = weight.shape[1]
    idxt = input_ids.T.astype(jnp.int32).reshape(_NW, -1, _CHUNK)
    out = _embedding_gather(idxt, weight)
    return out.reshape(H, B, D).transpose(1, 0, 2)


# speedup vs baseline: 1.7615x; 1.0013x over previous
import functools

import jax
import jax.numpy as jnp
from jax import lax
from jax.experimental import pallas as pl
from jax.experimental.pallas import tpu as pltpu
from jax.experimental.pallas import tpu_sc as plsc

_NC = 2
_NS = 16
_NW = _NC * _NS
_CHUNK = 64
_NBUF = 10


@jax.jit
def _embedding_gather(idx, weight):
    n_chunks = idx.shape[1]
    D = weight.shape[1]
    R = _NW * n_chunks * _CHUNK
    rows_per_w = n_chunks * _CHUNK
    mesh = plsc.VectorSubcoreMesh(core_axis_name="c", subcore_axis_name="s")

    @functools.partial(
        pl.kernel,
        mesh=mesh,
        out_type=jax.ShapeDtypeStruct((R, D), jnp.float32),
        scratch_types=[
            pltpu.VMEM((n_chunks, _CHUNK), jnp.int32),
            pltpu.VMEM((_NBUF, _CHUNK, D), jnp.float32),
            pltpu.SemaphoreType.DMA((_NBUF,)),
            pltpu.SemaphoreType.DMA((_NBUF,)),
        ],
    )
    def k(idx_hbm, table_hbm, out_hbm, idx_v, rows_v, gsem, ssem):
        wid = lax.axis_index("s") * _NC + lax.axis_index("c")
        base = wid * rows_per_w

        pltpu.sync_copy(idx_hbm.at[wid], idx_v)

        def gather(j, b):
            pltpu.async_copy(table_hbm.at[idx_v.at[j]], rows_v.at[b], gsem.at[b])

        def gather_wait(j, b):
            pltpu.make_async_copy(
                table_hbm.at[idx_v.at[j]], rows_v.at[b], gsem.at[b]
            ).wait()

        def scatter(j, b):
            pltpu.async_copy(
                rows_v.at[b], out_hbm.at[pl.ds(base + j * _CHUNK, _CHUNK)], ssem.at[b]
            )

        def scatter_wait(b):
            pltpu.make_async_copy(
                rows_v.at[b], out_hbm.at[pl.ds(base, _CHUNK)], ssem.at[b]
            ).wait()

        for b in range(_NBUF - 1):
            gather(b, b)

        @pl.loop(0, n_chunks, step=_NBUF)
        def _(j0):
            for b in range(_NBUF):
                j = j0 + b
                pb = (b - 1) % _NBUF
                gather_wait(j, b)
                scatter(j, b)
                @pl.when(j > 0)
                def _():
                    scatter_wait(pb)

                @pl.when(j + _NBUF - 1 < n_chunks)
                def _():
                    gather(j + _NBUF - 1, pb)

        scatter_wait((n_chunks - 1) % _NBUF)

    return k(idx, weight)


def kernel(input_ids, weight):
    B, H = input_ids.shape
    D = weight.shape[1]
    idxt = input_ids.T.astype(jnp.int32).reshape(_NW, -1, _CHUNK)
    out = _embedding_gather(idxt, weight)
    return out.reshape(H, B, D).transpose(1, 0, 2)
